# baseline (speedup 1.0000x reference)
import functools

import jax
import jax.numpy as jnp
from jax import lax
from jax.experimental import pallas as pl
from jax.experimental.pallas import tpu as pltpu
from jax.experimental.pallas import tpu_sc as plsc

N0, N1, N2 = 10000, 5000, 2500
D, HID, OUT = 128, 256, 64
K1 = 5120
M1 = 2560
K2 = 2560
M2 = 2560
BR = 256


def _ln(h, g, b):
    mu = jnp.mean(h, axis=-1, keepdims=True)
    var = jnp.mean((h - mu) * (h - mu), axis=-1, keepdims=True)
    return (h - mu) * jax.lax.rsqrt(var + 1e-5) * g + b


def _prep_body(x_ref, t_ref, u_ref, v_ref):
    x = x_ref[...]
    r = jnp.maximum(x, 0.0) + 1e-7
    z = t_ref[0, 0] * r
    c = jnp.max(z, axis=0, keepdims=True)
    v = jnp.exp(z - c)
    u_ref[...] = v * r
    v_ref[...] = v


def _prep(x_pad, t):
    n = x_pad.shape[0]
    return pl.pallas_call(
        _prep_body,
        out_shape=(
            jax.ShapeDtypeStruct((n, D), jnp.float32),
            jax.ShapeDtypeStruct((n, D), jnp.float32),
        ),
    )(x_pad, t.reshape(1, 1))


def _layer_body(a_ref, p_ref, xd_ref, w1_ref, b1_ref, g1_ref, be1_ref,
                w2_ref, b2_ref, ng_ref, nb_ref, o_ref, *, post):
    a = a_ref[...]
    uv = jnp.dot(a, p_ref[...], preferred_element_type=jnp.float32)
    agg = uv[:, :D] / (uv[:, D:] + 1e-16)
    h0 = agg + xd_ref[...]
    h = jnp.dot(h0, w1_ref[...], preferred_element_type=jnp.float32) + b1_ref[...]
    h = jnp.maximum(_ln(h, g1_ref[...], be1_ref[...]), 0.0)
    y = jnp.dot(h, w2_ref[...], preferred_element_type=jnp.float32) + b2_ref[...]
    if post == "gelu_ln":
        o_ref[...] = jax.nn.gelu(_ln(y, ng_ref[...], nb_ref[...]))
    else:
        m = jnp.max(y, axis=-1, keepdims=True)
        e = jnp.exp(y - m)
        o_ref[...] = y - m - jnp.log(jnp.sum(e, axis=-1, keepdims=True))


def _layer(A, P, xd, W1, b1, g1, be1, W2, b2, ng, nb, post, dout):
    m, k = A.shape
    body = functools.partial(_layer_body, post=post)
    grid = (m // BR,)
    return pl.pallas_call(
        body,
        grid=grid,
        in_specs=[
            pl.BlockSpec((BR, k), lambda i: (i, 0)),
            pl.BlockSpec((k, 2 * D), lambda i: (0, 0)),
            pl.BlockSpec((BR, D), lambda i: (i, 0)),
            pl.BlockSpec((D, HID), lambda i: (0, 0)),
            pl.BlockSpec((1, HID), lambda i: (0, 0)),
            pl.BlockSpec((1, HID), lambda i: (0, 0)),
            pl.BlockSpec((1, HID), lambda i: (0, 0)),
            pl.BlockSpec((HID, dout), lambda i: (0, 0)),
            pl.BlockSpec((1, dout), lambda i: (0, 0)),
            pl.BlockSpec((1, dout), lambda i: (0, 0)),
            pl.BlockSpec((1, dout), lambda i: (0, 0)),
        ],
        out_specs=pl.BlockSpec((BR, dout), lambda i: (i, 0)),
        out_shape=jax.ShapeDtypeStruct((m, dout), jnp.float32),
    )(A, P, xd, W1, b1.reshape(1, -1), g1.reshape(1, -1), be1.reshape(1, -1),
      W2, b2.reshape(1, -1), ng.reshape(1, -1), nb.reshape(1, -1))


def _sc_counts(src_arr, dst_arr, m_dst, k_src, rounds, unroll, nchunks):
    e = src_arr.shape[0]
    info = plsc.get_sparse_core_info()
    nc, ns = info.num_cores, info.num_subcores
    ec = e // ns
    ch = ec // nchunks
    r_rows = m_dst // (nc * rounds)
    slab_len = r_rows * k_src
    dump = slab_len
    stripe = slab_len // ns
    zb = 10240
    assert ec % (16 * unroll) == 0 and ch % (16 * unroll) == 0
    assert stripe % zb == 0 and slab_len % ns == 0

    def body(src_hbm, dst_hbm, out_hbm, e0, e1, idx, ones, zbuf, slab):
        c = lax.axis_index("c")
        s = lax.axis_index("s")
        pltpu.sync_copy(src_hbm.at[pl.ds(s * ec, ec)], e0)
        pltpu.sync_copy(dst_hbm.at[pl.ds(s * ec, ec)], e1)

        def prep_body(i, _):
            for u in range(unroll):
                b = (i * unroll + u) * 16
                e1[pl.ds(b, 16)] = e1[pl.ds(b, 16)] * k_src + e0[pl.ds(b, 16)]
            return 0

        lax.fori_loop(0, ec // (16 * unroll), prep_body, 0)

        def ones_body(i, _):
            for u in range(unroll):
                b = (i * unroll + u) * 16
                ones[pl.ds(b, 16)] = jnp.full((16,), 1.0, jnp.float32)
            return 0

        lax.fori_loop(0, ch // (16 * unroll), ones_body, 0)

        def zb_body(i, _):
            for u in range(8):
                zbuf[pl.ds((i * 8 + u) * 16, 16)] = jnp.zeros((16,), jnp.float32)
            return 0

        lax.fori_loop(0, zb // 128, zb_body, 0)

        for r in range(rounds):
            for j in range(stripe // zb):
                pltpu.sync_copy(zbuf, slab.at[pl.ds(s * stripe + j * zb, zb)])
            plsc.subcore_barrier()

            base = (c * rounds + r) * slab_len
            for q in range(nchunks):
                def idx_body(i, _):
                    for u in range(unroll):
                        b = (i * unroll + u) * 16
                        t = e1[pl.ds(q * ch + b, 16)] - base
                        ok = (t >= 0) & (t < slab_len)
                        idx[pl.ds(b, 16)] = jnp.where(ok, t, dump)
                    return 0

                lax.fori_loop(0, ch // (16 * unroll), idx_body, 0)
                pltpu.sync_copy(ones, slab.at[idx], add=True)
            plsc.subcore_barrier()
            pltpu.sync_copy(
                slab.at[pl.ds(s * stripe, stripe)],
                out_hbm.at[pl.ds(base + s * stripe, stripe)])
            plsc.subcore_barrier()

    mesh = plsc.VectorSubcoreMesh(core_axis_name="c", subcore_axis_name="s")
    flat = pl.kernel(
        body,
        out_type=jax.ShapeDtypeStruct((m_dst * k_src,), jnp.float32),
        mesh=mesh,
        scratch_types=[
            pltpu.VMEM((ec,), jnp.int32),
            pltpu.VMEM((ec,), jnp.int32),
            pltpu.VMEM((ch,), jnp.int32),
            pltpu.VMEM((ch,), jnp.float32),
            pltpu.VMEM((zb,), jnp.float32),
            pltpu.VMEM_SHARED((slab_len + 8,), jnp.float32),
        ],
    )(src_arr, dst_arr)
    return flat.reshape(m_dst, k_src)


def _build_counts(edge_index, m_dst, k_src, rounds, unroll, nchunks):
    return _sc_counts(edge_index[0], edge_index[1], m_dst, k_src,
                      rounds, unroll, nchunks)


def kernel(x, edge_index1, edge_index2, t1, W1a, b1a, g1a, be1a, W1b, b1b,
           ng, nb, t2, W2a, b2a, g2a, be2a, W2b, b2b):
    x_src = jnp.concatenate(
        [x[:N1], jnp.zeros((K1 - N1, D), jnp.float32)], axis=0)
    u1, v1 = _prep(x_src, t1)
    P1 = jnp.concatenate([u1, v1], axis=1)
    A1 = _build_counts(edge_index1, M1, K1, rounds=8, unroll=5, nchunks=2)
    hg = _layer(A1, P1, x[:M1], W1a, b1a, g1a, be1a, W1b, b1b, ng, nb,
                "gelu_ln", D)
    u2, v2 = _prep(hg, t2)
    P2 = jnp.concatenate([u2, v2], axis=1)
    A2 = _build_counts(edge_index2, M2, K2, rounds=4, unroll=5, nchunks=1)
    out = _layer(A2, P2, hg, W2a, b2a, g2a, be2a, W2b, b2b,
                 jnp.zeros((OUT,), jnp.float32), jnp.zeros((OUT,), jnp.float32),
                 "log_softmax", OUT)
    return out[:N2]


# speedup vs baseline: 12.5144x; 12.5144x over previous
import functools

import jax
import jax.numpy as jnp
from jax import lax
from jax.experimental import pallas as pl
from jax.experimental.pallas import tpu as pltpu
from jax.experimental.pallas import tpu_sc as plsc

N0, N1, N2 = 10000, 5000, 2500
D, HID, OUT = 128, 256, 64
K1 = 5120
M1 = 2560
K2 = 2560
M2 = 2560
BR = 256


def _ln(h, g, b):
    mu = jnp.mean(h, axis=-1, keepdims=True)
    var = jnp.mean((h - mu) * (h - mu), axis=-1, keepdims=True)
    return (h - mu) * jax.lax.rsqrt(var + 1e-5) * g + b


def _prep_body(x_ref, t_ref, u_ref, v_ref):
    x = x_ref[...]
    r = jnp.maximum(x, 0.0) + 1e-7
    z = t_ref[0, 0] * r
    c = jnp.max(z, axis=0, keepdims=True)
    v = jnp.exp(z - c)
    u_ref[...] = v * r
    v_ref[...] = v


def _prep(x_pad, t):
    n = x_pad.shape[0]
    return pl.pallas_call(
        _prep_body,
        out_shape=(
            jax.ShapeDtypeStruct((n, D), jnp.float32),
            jax.ShapeDtypeStruct((n, D), jnp.float32),
        ),
    )(x_pad, t.reshape(1, 1))


def _layer_body(a_ref, p_ref, xd_ref, w1_ref, b1_ref, g1_ref, be1_ref,
                w2_ref, b2_ref, ng_ref, nb_ref, o_ref, *, post):
    a = a_ref[...]
    uv = jnp.dot(a, p_ref[...], preferred_element_type=jnp.float32)
    agg = uv[:, :D] / (uv[:, D:] + 1e-16)
    h0 = agg + xd_ref[...]
    h = jnp.dot(h0, w1_ref[...], preferred_element_type=jnp.float32) + b1_ref[...]
    h = jnp.maximum(_ln(h, g1_ref[...], be1_ref[...]), 0.0)
    y = jnp.dot(h, w2_ref[...], preferred_element_type=jnp.float32) + b2_ref[...]
    if post == "gelu_ln":
        o_ref[...] = jax.nn.gelu(_ln(y, ng_ref[...], nb_ref[...]))
    else:
        m = jnp.max(y, axis=-1, keepdims=True)
        e = jnp.exp(y - m)
        o_ref[...] = y - m - jnp.log(jnp.sum(e, axis=-1, keepdims=True))


def _layer(A, P, xd, W1, b1, g1, be1, W2, b2, ng, nb, post, dout):
    m, k = A.shape
    body = functools.partial(_layer_body, post=post)
    grid = (m // BR,)
    return pl.pallas_call(
        body,
        grid=grid,
        in_specs=[
            pl.BlockSpec((BR, k), lambda i: (i, 0)),
            pl.BlockSpec((k, 2 * D), lambda i: (0, 0)),
            pl.BlockSpec((BR, D), lambda i: (i, 0)),
            pl.BlockSpec((D, HID), lambda i: (0, 0)),
            pl.BlockSpec((1, HID), lambda i: (0, 0)),
            pl.BlockSpec((1, HID), lambda i: (0, 0)),
            pl.BlockSpec((1, HID), lambda i: (0, 0)),
            pl.BlockSpec((HID, dout), lambda i: (0, 0)),
            pl.BlockSpec((1, dout), lambda i: (0, 0)),
            pl.BlockSpec((1, dout), lambda i: (0, 0)),
            pl.BlockSpec((1, dout), lambda i: (0, 0)),
        ],
        out_specs=pl.BlockSpec((BR, dout), lambda i: (i, 0)),
        out_shape=jax.ShapeDtypeStruct((m, dout), jnp.float32),
    )(A, P, xd, W1, b1.reshape(1, -1), g1.reshape(1, -1), be1.reshape(1, -1),
      W2, b2.reshape(1, -1), ng.reshape(1, -1), nb.reshape(1, -1))


def _sc_counts(src_arr, dst_arr, m_dst, k_src, rounds, unroll, nchunks):
    e = src_arr.shape[0]
    info = plsc.get_sparse_core_info()
    nc, ns = info.num_cores, info.num_subcores
    ec = e // ns
    ch = ec // nchunks
    r_rows = m_dst // (nc * rounds)
    slab_len = r_rows * k_src
    dump_n = 4096
    stripe = slab_len // ns
    zb = 10240
    assert ec % (16 * unroll) == 0 and ch % (16 * unroll) == 0
    assert stripe % zb == 0 and slab_len % ns == 0

    def body(src_hbm, dst_hbm, out_hbm, e0, e1, idx, ones, zbuf, slab):
        c = lax.axis_index("c")
        s = lax.axis_index("s")
        pltpu.sync_copy(src_hbm.at[pl.ds(s * ec, ec)], e0)
        pltpu.sync_copy(dst_hbm.at[pl.ds(s * ec, ec)], e1)

        def prep_body(i, _):
            for u in range(unroll):
                b = (i * unroll + u) * 16
                e1[pl.ds(b, 16)] = e1[pl.ds(b, 16)] * k_src + e0[pl.ds(b, 16)]
            return 0

        lax.fori_loop(0, ec // (16 * unroll), prep_body, 0)

        def ones_body(i, _):
            for u in range(unroll):
                b = (i * unroll + u) * 16
                ones[pl.ds(b, 16)] = jnp.full((16,), 1.0, jnp.float32)
            return 0

        lax.fori_loop(0, ch // (16 * unroll), ones_body, 0)

        def zb_body(i, _):
            for u in range(8):
                zbuf[pl.ds((i * 8 + u) * 16, 16)] = jnp.zeros((16,), jnp.float32)
            return 0

        lax.fori_loop(0, zb // 128, zb_body, 0)

        for r in range(rounds):
            for j in range(stripe // zb):
                pltpu.sync_copy(zbuf, slab.at[pl.ds(s * stripe + j * zb, zb)])
            plsc.subcore_barrier()

            base = (c * rounds + r) * slab_len
            for q in range(nchunks):
                def idx_body(i, _):
                    for u in range(unroll):
                        b = (i * unroll + u) * 16
                        t = e1[pl.ds(q * ch + b, 16)] - base
                        ok = (t >= 0) & (t < slab_len)
                        dmp = slab_len + (t & (dump_n - 1))
                        idx[pl.ds(b, 16)] = jnp.where(ok, t, dmp)
                    return 0

                lax.fori_loop(0, ch // (16 * unroll), idx_body, 0)
                pltpu.sync_copy(ones, slab.at[idx], add=True)
            plsc.subcore_barrier()
            pltpu.sync_copy(
                slab.at[pl.ds(s * stripe, stripe)],
                out_hbm.at[pl.ds(base + s * stripe, stripe)])
            plsc.subcore_barrier()

    mesh = plsc.VectorSubcoreMesh(core_axis_name="c", subcore_axis_name="s")
    flat = pl.kernel(
        body,
        out_type=jax.ShapeDtypeStruct((m_dst * k_src,), jnp.float32),
        mesh=mesh,
        scratch_types=[
            pltpu.VMEM((ec,), jnp.int32),
            pltpu.VMEM((ec,), jnp.int32),
            pltpu.VMEM((ch,), jnp.int32),
            pltpu.VMEM((ch,), jnp.float32),
            pltpu.VMEM((zb,), jnp.float32),
            pltpu.VMEM_SHARED((slab_len + dump_n,), jnp.float32),
        ],
    )(src_arr, dst_arr)
    return flat.reshape(m_dst, k_src)


def _build_counts(edge_index, m_dst, k_src, rounds, unroll, nchunks):
    return _sc_counts(edge_index[0], edge_index[1], m_dst, k_src,
                      rounds, unroll, nchunks)


def kernel(x, edge_index1, edge_index2, t1, W1a, b1a, g1a, be1a, W1b, b1b,
           ng, nb, t2, W2a, b2a, g2a, be2a, W2b, b2b):
    x_src = jnp.concatenate(
        [x[:N1], jnp.zeros((K1 - N1, D), jnp.float32)], axis=0)
    u1, v1 = _prep(x_src, t1)
    P1 = jnp.concatenate([u1, v1], axis=1)
    A1 = _build_counts(edge_index1, M1, K1, rounds=8, unroll=5, nchunks=2)
    hg = _layer(A1, P1, x[:M1], W1a, b1a, g1a, be1a, W1b, b1b, ng, nb,
                "gelu_ln", D)
    u2, v2 = _prep(hg, t2)
    P2 = jnp.concatenate([u2, v2], axis=1)
    A2 = _build_counts(edge_index2, M2, K2, rounds=4, unroll=5, nchunks=1)
    out = _layer(A2, P2, hg, W2a, b2a, g2a, be2a, W2b, b2b,
                 jnp.zeros((OUT,), jnp.float32), jnp.zeros((OUT,), jnp.float32),
                 "log_softmax", OUT)
    return out[:N2]
